# baseline (speedup 1.0000x reference)
import functools

import jax
import jax.numpy as jnp
import numpy as np
from jax import lax
from jax.experimental import pallas as pl
from jax.experimental.pallas import tpu as pltpu
from jax.experimental.pallas import tpu_sc as plsc

EPS = 0.5
NC, NS = 2, 16
NW = NC * NS
LANES = 16


def _mlp1_body(x_ref, w_ref, b_ref, y_ref):
    y_ref[...] = (
        jnp.dot(x_ref[...], w_ref[...], preferred_element_type=jnp.float32)
        + 0.5 * b_ref[...]
    )


def _mlp1(x, w1, b1):
    n, d_in = x.shape
    d_h = w1.shape[1]
    blk = 2000
    return pl.pallas_call(
        _mlp1_body,
        grid=(n // blk,),
        in_specs=[
            pl.BlockSpec((blk, d_in), lambda i: (i, 0)),
            pl.BlockSpec((d_in, d_h), lambda i: (0, 0)),
            pl.BlockSpec((1, d_h), lambda i: (0, 0)),
        ],
        out_specs=pl.BlockSpec((blk, d_h), lambda i: (i, 0)),
        out_shape=jax.ShapeDtypeStruct((n, d_h), jnp.float32),
    )(x, w1, b1.reshape(1, -1))


def _mlp2_body(z_ref, w_ref, b_ref, o_ref):
    h = jnp.maximum(z_ref[...], 0.0)
    o_ref[...] = EPS * (
        jnp.dot(h, w_ref[...], preferred_element_type=jnp.float32) + b_ref[...]
    )


def _mlp2_body_aliased(z_ref, w_ref, b_ref, oprev_ref, o_ref):
    del oprev_ref
    _mlp2_body(z_ref, w_ref, b_ref, o_ref)


MLP2_BLK = 4000


def _mlp2_slice(z, w2, b2, e_total, base_blk, out_prev):
    e_s, d_h = z.shape
    d_out = w2.shape[1]
    blk = MLP2_BLK
    in_specs = [
        pl.BlockSpec((blk, d_h), lambda i: (i, 0)),
        pl.BlockSpec((d_h, d_out), lambda i: (0, 0)),
        pl.BlockSpec((1, d_out), lambda i: (0, 0)),
    ]
    args = [z, w2, b2.reshape(1, -1)]
    kwargs = {}
    body = _mlp2_body
    if out_prev is not None:
        in_specs.append(pl.BlockSpec(memory_space=pl.ANY))
        args.append(out_prev)
        kwargs["input_output_aliases"] = {3: 0}
        body = _mlp2_body_aliased
    return pl.pallas_call(
        body,
        grid=(e_s // blk,),
        in_specs=in_specs,
        out_specs=pl.BlockSpec((blk, d_out), lambda i: (base_blk + i, 0)),
        out_shape=jax.ShapeDtypeStruct((e_total, d_out), jnp.float32),
        **kwargs,
    )(*args)


CHUNK = 40
NBUF = 3


def _gather_sum_body(chunks_per_w, d_h, e_per_w,
                     y_hbm, idx2_hbm, z_hbm,
                     idx_all, y_sh, rows0, rows1, rows2,
                     gs0, gs1, gs2, ws0, ws1, ws2):
    sid = lax.axis_index("s")
    wid = sid * NC + lax.axis_index("c")
    base = wid * e_per_w
    rows = (rows0, rows1, rows2)
    gsem = (gs0, gs1, gs2)
    wsem = (ws0, ws1, ws2)
    ch = chunks_per_w

    n_nodes = y_hbm.shape[0]
    stripe = (n_nodes // NS) & ~7
    rem = n_nodes - NS * stripe
    pltpu.sync_copy(
        y_hbm.at[pl.ds(pl.multiple_of(sid * stripe, 8), stripe)],
        y_sh.at[pl.ds(pl.multiple_of(sid * stripe, 8), stripe)],
    )
    if rem:
        @pl.when(sid == 0)
        def _stage_rem():
            pltpu.sync_copy(
                y_hbm.at[pl.ds(NS * stripe, rem)],
                y_sh.at[pl.ds(NS * stripe, rem)],
            )
    pltpu.sync_copy(idx2_hbm.at[wid], idx_all)
    plsc.subcore_barrier()

    def issue(cur, b):
        pltpu.async_copy(y_sh.at[idx_all.at[cur]], rows[b], gsem[b])

    def wait_gather(cur, b):
        pltpu.make_async_copy(y_sh.at[idx_all.at[cur]], rows[b], gsem[b]).wait()

    def wait_write(b):
        pltpu.make_async_copy(
            rows[b].at[pl.ds(0, CHUNK)], z_hbm.at[pl.ds(0, CHUNK)], wsem[b]
        ).wait()

    def process(cur, b):
        wait_gather(cur, b)

        def add_row(i, c):
            for v in range(d_h // LANES):
                sl = pl.ds(v * LANES, LANES)
                plsc.addupdate(rows[b].at[i, sl], rows[b][CHUNK + i, sl])
            return c

        lax.fori_loop(0, CHUNK, add_row, 0, unroll=4)
        off = base + cur * CHUNK
        pltpu.async_copy(
            rows[b].at[pl.ds(0, CHUNK)], z_hbm.at[pl.ds(off, CHUNK)], wsem[b]
        )

    issue(0, 0)
    issue(1, 1)

    def body(k, carry):
        g = 3 * k
        for b in range(3):
            cur = g + b
            nxt = cur + 2
            bn = (b + 2) % 3

            @pl.when(nxt < ch)
            def _issue():
                if b == 0:
                    @pl.when(k > 0)
                    def _w():
                        wait_write(bn)
                else:
                    wait_write(bn)
                issue(nxt, bn)

            process(cur, b)
        return carry

    lax.fori_loop(0, ch // 3, body, 0)
    for cur in range(3 * (ch // 3), ch):
        b = cur % 3
        nxt = cur + 2
        if nxt < ch:
            wait_write((b + 2) % 3)
            issue(nxt, (b + 2) % 3)
        process(cur, b)
    wait_write(0)
    wait_write(1)
    wait_write(2)


def _gather_sum(y, idx2):
    n, d_h = y.shape
    nw, ch, twoc = idx2.shape
    e = nw * ch * CHUNK
    e_per_w = e // NW

    mesh = plsc.VectorSubcoreMesh(
        core_axis_name="c", subcore_axis_name="s", num_cores=NC, num_subcores=NS
    )
    body = functools.partial(_gather_sum_body, ch, d_h, e_per_w)
    return pl.kernel(
        body,
        out_type=jax.ShapeDtypeStruct((e, d_h), jnp.float32),
        mesh=mesh,
        scratch_types=[
            pltpu.VMEM((ch, twoc), jnp.int32),
            pltpu.VMEM_SHARED((n, d_h), jnp.float32),
            pltpu.VMEM((2 * CHUNK, d_h), jnp.float32),
            pltpu.VMEM((2 * CHUNK, d_h), jnp.float32),
            pltpu.VMEM((2 * CHUNK, d_h), jnp.float32),
            pltpu.SemaphoreType.DMA,
            pltpu.SemaphoreType.DMA,
            pltpu.SemaphoreType.DMA,
            pltpu.SemaphoreType.DMA,
            pltpu.SemaphoreType.DMA,
            pltpu.SemaphoreType.DMA,
        ],
    )(y, idx2)


NSLICE = 5


def kernel(x, edge_index, W1, b1, W2, b2):
    y = _mlp1(x, W1, b1)
    e = edge_index.shape[1]
    e_s = e // NSLICE
    ch = e_s // (NW * CHUNK)
    out = None
    for s in range(NSLICE):
        lo = s * e_s
        idx2 = jnp.stack(
            [
                lax.slice(edge_index[0], (lo,), (lo + e_s,)).reshape(NW, ch, CHUNK),
                lax.slice(edge_index[1], (lo,), (lo + e_s,)).reshape(NW, ch, CHUNK),
            ],
            axis=2,
        ).reshape(NW, ch, 2 * CHUNK)
        z = _gather_sum(y, idx2)
        out = _mlp2_slice(z, W2, b2, e, s * (e_s // MLP2_BLK), out)
    return out


# speedup vs baseline: 1.0315x; 1.0315x over previous
import functools

import jax
import jax.numpy as jnp
import numpy as np
from jax import lax
from jax.experimental import pallas as pl
from jax.experimental.pallas import tpu as pltpu
from jax.experimental.pallas import tpu_sc as plsc

EPS = 0.5
NC, NS = 2, 16
NW = NC * NS
LANES = 16


def _mlp1_body(x_ref, w_ref, b_ref, y_ref):
    y_ref[...] = (
        jnp.dot(x_ref[...], w_ref[...], preferred_element_type=jnp.float32)
        + 0.5 * b_ref[...]
    )


def _mlp1(x, w1, b1):
    n, d_in = x.shape
    d_h = w1.shape[1]
    blk = 2000
    return pl.pallas_call(
        _mlp1_body,
        grid=(n // blk,),
        in_specs=[
            pl.BlockSpec((blk, d_in), lambda i: (i, 0)),
            pl.BlockSpec((d_in, d_h), lambda i: (0, 0)),
            pl.BlockSpec((1, d_h), lambda i: (0, 0)),
        ],
        out_specs=pl.BlockSpec((blk, d_h), lambda i: (i, 0)),
        out_shape=jax.ShapeDtypeStruct((n, d_h), jnp.float32),
    )(x, w1, b1.reshape(1, -1))


def _mlp2_body(z_ref, w_ref, b_ref, o_ref):
    h = jnp.maximum(z_ref[...], 0.0)
    o_ref[...] = EPS * (
        jnp.dot(h, w_ref[...], preferred_element_type=jnp.float32) + b_ref[...]
    )


def _mlp2_body_aliased(z_ref, w_ref, b_ref, oprev_ref, o_ref):
    del oprev_ref
    _mlp2_body(z_ref, w_ref, b_ref, o_ref)


MLP2_BLK = 4000


def _mlp2_slice(z, w2, b2, e_total, base_blk, out_prev):
    e_s, d_h = z.shape
    d_out = w2.shape[1]
    blk = MLP2_BLK
    in_specs = [
        pl.BlockSpec((blk, d_h), lambda i: (i, 0)),
        pl.BlockSpec((d_h, d_out), lambda i: (0, 0)),
        pl.BlockSpec((1, d_out), lambda i: (0, 0)),
    ]
    args = [z, w2, b2.reshape(1, -1)]
    kwargs = {}
    body = _mlp2_body
    if out_prev is not None:
        in_specs.append(pl.BlockSpec(memory_space=pl.ANY))
        args.append(out_prev)
        kwargs["input_output_aliases"] = {3: 0}
        body = _mlp2_body_aliased
    return pl.pallas_call(
        body,
        grid=(e_s // blk,),
        in_specs=in_specs,
        out_specs=pl.BlockSpec((blk, d_out), lambda i: (base_blk + i, 0)),
        out_shape=jax.ShapeDtypeStruct((e_total, d_out), jnp.float32),
        **kwargs,
    )(*args)


CHUNK = 40
NBUF = 3


def _gather_sum_body(chunks_per_w, d_h, e_per_w,
                     y_hbm, idx2_hbm, z_hbm,
                     idx_all, y_sh, rows0, rows1, rows2,
                     gs0, gs1, gs2, ws0, ws1, ws2):
    sid = lax.axis_index("s")
    wid = sid * NC + lax.axis_index("c")
    base = wid * e_per_w
    rows = (rows0, rows1, rows2)
    gsem = (gs0, gs1, gs2)
    wsem = (ws0, ws1, ws2)
    ch = chunks_per_w

    n_nodes = y_hbm.shape[0]
    stripe = (n_nodes // NS) & ~7
    rem = n_nodes - NS * stripe
    pltpu.sync_copy(
        y_hbm.at[pl.ds(pl.multiple_of(sid * stripe, 8), stripe)],
        y_sh.at[pl.ds(pl.multiple_of(sid * stripe, 8), stripe)],
    )
    if rem:
        @pl.when(sid == 0)
        def _stage_rem():
            pltpu.sync_copy(
                y_hbm.at[pl.ds(NS * stripe, rem)],
                y_sh.at[pl.ds(NS * stripe, rem)],
            )
    pltpu.sync_copy(idx2_hbm.at[wid], idx_all)
    plsc.subcore_barrier()

    def issue(cur, b):
        pltpu.async_copy(y_sh.at[idx_all.at[cur]], rows[b], gsem[b])

    def wait_gather(cur, b):
        pltpu.make_async_copy(y_sh.at[idx_all.at[cur]], rows[b], gsem[b]).wait()

    def wait_write(b):
        pltpu.make_async_copy(
            rows[b].at[pl.ds(0, CHUNK)], z_hbm.at[pl.ds(0, CHUNK)], wsem[b]
        ).wait()

    def process(cur, b):
        wait_gather(cur, b)

        def add_row(i, c):
            for v in range(d_h // LANES):
                sl = pl.ds(v * LANES, LANES)
                plsc.addupdate(rows[b].at[i, sl], rows[b][CHUNK + i, sl])
            return c

        lax.fori_loop(0, CHUNK, add_row, 0, unroll=4)
        off = base + cur * CHUNK
        pltpu.async_copy(
            rows[b].at[pl.ds(0, CHUNK)], z_hbm.at[pl.ds(off, CHUNK)], wsem[b]
        )

    issue(0, 0)
    issue(1, 1)

    def body(k, carry):
        g = 3 * k
        for b in range(3):
            cur = g + b
            nxt = cur + 2
            bn = (b + 2) % 3

            @pl.when(nxt < ch)
            def _issue():
                if b == 0:
                    @pl.when(k > 0)
                    def _w():
                        wait_write(bn)
                else:
                    wait_write(bn)
                issue(nxt, bn)

            process(cur, b)
        return carry

    lax.fori_loop(0, ch // 3, body, 0)
    for cur in range(3 * (ch // 3), ch):
        b = cur % 3
        nxt = cur + 2
        if nxt < ch:
            wait_write((b + 2) % 3)
            issue(nxt, (b + 2) % 3)
        process(cur, b)
    wait_write(0)
    wait_write(1)
    wait_write(2)


def _gather_sum(y, idx2):
    n, d_h = y.shape
    nw, ch, twoc = idx2.shape
    e = nw * ch * CHUNK
    e_per_w = e // NW

    mesh = plsc.VectorSubcoreMesh(
        core_axis_name="c", subcore_axis_name="s", num_cores=NC, num_subcores=NS
    )
    body = functools.partial(_gather_sum_body, ch, d_h, e_per_w)
    return pl.kernel(
        body,
        out_type=jax.ShapeDtypeStruct((e, d_h), jnp.float32),
        mesh=mesh,
        scratch_types=[
            pltpu.VMEM((ch, twoc), jnp.int32),
            pltpu.VMEM_SHARED((n, d_h), jnp.float32),
            pltpu.VMEM((2 * CHUNK, d_h), jnp.float32),
            pltpu.VMEM((2 * CHUNK, d_h), jnp.float32),
            pltpu.VMEM((2 * CHUNK, d_h), jnp.float32),
            pltpu.SemaphoreType.DMA,
            pltpu.SemaphoreType.DMA,
            pltpu.SemaphoreType.DMA,
            pltpu.SemaphoreType.DMA,
            pltpu.SemaphoreType.DMA,
            pltpu.SemaphoreType.DMA,
        ],
    )(y, idx2)


NSLICE = 2


def kernel(x, edge_index, W1, b1, W2, b2):
    y = _mlp1(x, W1, b1)
    e = edge_index.shape[1]
    e_s = e // NSLICE
    ch = e_s // (NW * CHUNK)
    out = None
    for s in range(NSLICE):
        lo = s * e_s
        idx2 = jnp.stack(
            [
                lax.slice(edge_index[0], (lo,), (lo + e_s,)).reshape(NW, ch, CHUNK),
                lax.slice(edge_index[1], (lo,), (lo + e_s,)).reshape(NW, ch, CHUNK),
            ],
            axis=2,
        ).reshape(NW, ch, 2 * CHUNK)
        z = _gather_sum(y, idx2)
        out = _mlp2_slice(z, W2, b2, e, s * (e_s // MLP2_BLK), out)
    return out


# speedup vs baseline: 1.0451x; 1.0132x over previous
import functools

import jax
import jax.numpy as jnp
import numpy as np
from jax import lax
from jax.experimental import pallas as pl
from jax.experimental.pallas import tpu as pltpu
from jax.experimental.pallas import tpu_sc as plsc

EPS = 0.5
NC, NS = 2, 16
NW = NC * NS
LANES = 16


def _mlp1_body(x_ref, w_ref, b_ref, y_ref):
    y_ref[...] = (
        jnp.dot(x_ref[...], w_ref[...], preferred_element_type=jnp.float32)
        + 0.5 * b_ref[...]
    )


def _mlp1(x, w1, b1):
    n, d_in = x.shape
    d_h = w1.shape[1]
    blk = 2000
    return pl.pallas_call(
        _mlp1_body,
        grid=(n // blk,),
        in_specs=[
            pl.BlockSpec((blk, d_in), lambda i: (i, 0)),
            pl.BlockSpec((d_in, d_h), lambda i: (0, 0)),
            pl.BlockSpec((1, d_h), lambda i: (0, 0)),
        ],
        out_specs=pl.BlockSpec((blk, d_h), lambda i: (i, 0)),
        out_shape=jax.ShapeDtypeStruct((n, d_h), jnp.float32),
    )(x, w1, b1.reshape(1, -1))


def _mlp2_body(z_ref, w_ref, b_ref, o_ref):
    h = jnp.maximum(z_ref[...], 0.0)
    o_ref[...] = EPS * (
        jnp.dot(h, w_ref[...], preferred_element_type=jnp.float32) + b_ref[...]
    )


def _mlp2_body_aliased(z_ref, w_ref, b_ref, oprev_ref, o_ref):
    del oprev_ref
    _mlp2_body(z_ref, w_ref, b_ref, o_ref)


MLP2_BLK = 8000


def _mlp2_slice(z, w2, b2, e_total, base_blk, out_prev):
    e_s, d_h = z.shape
    d_out = w2.shape[1]
    blk = MLP2_BLK
    in_specs = [
        pl.BlockSpec((blk, d_h), lambda i: (i, 0)),
        pl.BlockSpec((d_h, d_out), lambda i: (0, 0)),
        pl.BlockSpec((1, d_out), lambda i: (0, 0)),
    ]
    args = [z, w2, b2.reshape(1, -1)]
    kwargs = {}
    body = _mlp2_body
    if out_prev is not None:
        in_specs.append(pl.BlockSpec(memory_space=pl.ANY))
        args.append(out_prev)
        kwargs["input_output_aliases"] = {3: 0}
        body = _mlp2_body_aliased
    return pl.pallas_call(
        body,
        grid=(e_s // blk,),
        in_specs=in_specs,
        out_specs=pl.BlockSpec((blk, d_out), lambda i: (base_blk + i, 0)),
        out_shape=jax.ShapeDtypeStruct((e_total, d_out), jnp.float32),
        **kwargs,
    )(*args)


CHUNK = 40
NBUF = 3


def _gather_sum_body(chunks_per_w, d_h, e_per_w,
                     y_hbm, idx2_hbm, z_hbm,
                     idx_all, y_sh, rows0, rows1, rows2,
                     gs0, gs1, gs2, ws0, ws1, ws2):
    sid = lax.axis_index("s")
    wid = sid * NC + lax.axis_index("c")
    base = wid * e_per_w
    rows = (rows0, rows1, rows2)
    gsem = (gs0, gs1, gs2)
    wsem = (ws0, ws1, ws2)
    ch = chunks_per_w

    n_nodes = y_hbm.shape[0]
    stripe = (n_nodes // NS) & ~7
    rem = n_nodes - NS * stripe
    pltpu.sync_copy(
        y_hbm.at[pl.ds(pl.multiple_of(sid * stripe, 8), stripe)],
        y_sh.at[pl.ds(pl.multiple_of(sid * stripe, 8), stripe)],
    )
    if rem:
        @pl.when(sid == 0)
        def _stage_rem():
            pltpu.sync_copy(
                y_hbm.at[pl.ds(NS * stripe, rem)],
                y_sh.at[pl.ds(NS * stripe, rem)],
            )
    pltpu.sync_copy(idx2_hbm.at[wid], idx_all)
    plsc.subcore_barrier()

    def issue(cur, b):
        pltpu.async_copy(y_sh.at[idx_all.at[cur]], rows[b], gsem[b])

    def wait_gather(cur, b):
        pltpu.make_async_copy(y_sh.at[idx_all.at[cur]], rows[b], gsem[b]).wait()

    def wait_write(b):
        pltpu.make_async_copy(
            rows[b].at[pl.ds(0, CHUNK)], z_hbm.at[pl.ds(0, CHUNK)], wsem[b]
        ).wait()

    def process(cur, b):
        wait_gather(cur, b)

        def add_row(i, c):
            for v in range(d_h // LANES):
                sl = pl.ds(v * LANES, LANES)
                plsc.addupdate(rows[b].at[i, sl], rows[b][CHUNK + i, sl])
            return c

        plsc.parallel_loop(0, CHUNK, 1, unroll=4)(
            lambda i: add_row(i, 0) and None
        )
        off = base + cur * CHUNK
        pltpu.async_copy(
            rows[b].at[pl.ds(0, CHUNK)], z_hbm.at[pl.ds(off, CHUNK)], wsem[b]
        )

    issue(0, 0)
    issue(1, 1)

    def body(k, carry):
        g = 3 * k
        for b in range(3):
            cur = g + b
            nxt = cur + 2
            bn = (b + 2) % 3

            @pl.when(nxt < ch)
            def _issue():
                if b == 0:
                    @pl.when(k > 0)
                    def _w():
                        wait_write(bn)
                else:
                    wait_write(bn)
                issue(nxt, bn)

            process(cur, b)
        return carry

    lax.fori_loop(0, ch // 3, body, 0)
    for cur in range(3 * (ch // 3), ch):
        b = cur % 3
        nxt = cur + 2
        if nxt < ch:
            wait_write((b + 2) % 3)
            issue(nxt, (b + 2) % 3)
        process(cur, b)
    wait_write(0)
    wait_write(1)
    wait_write(2)


def _gather_sum(y, idx2):
    n, d_h = y.shape
    nw, ch, twoc = idx2.shape
    e = nw * ch * CHUNK
    e_per_w = e // NW

    mesh = plsc.VectorSubcoreMesh(
        core_axis_name="c", subcore_axis_name="s", num_cores=NC, num_subcores=NS
    )
    body = functools.partial(_gather_sum_body, ch, d_h, e_per_w)
    return pl.kernel(
        body,
        out_type=jax.ShapeDtypeStruct((e, d_h), jnp.float32),
        mesh=mesh,
        scratch_types=[
            pltpu.VMEM((ch, twoc), jnp.int32),
            pltpu.VMEM_SHARED((n, d_h), jnp.float32),
            pltpu.VMEM((2 * CHUNK, d_h), jnp.float32),
            pltpu.VMEM((2 * CHUNK, d_h), jnp.float32),
            pltpu.VMEM((2 * CHUNK, d_h), jnp.float32),
            pltpu.SemaphoreType.DMA,
            pltpu.SemaphoreType.DMA,
            pltpu.SemaphoreType.DMA,
            pltpu.SemaphoreType.DMA,
            pltpu.SemaphoreType.DMA,
            pltpu.SemaphoreType.DMA,
        ],
    )(y, idx2)


NSLICE = 2


def kernel(x, edge_index, W1, b1, W2, b2):
    y = _mlp1(x, W1, b1)
    e = edge_index.shape[1]
    e_s = e // NSLICE
    ch = e_s // (NW * CHUNK)
    out = None
    for s in range(NSLICE):
        lo = s * e_s
        idx2 = jnp.stack(
            [
                lax.slice(edge_index[0], (lo,), (lo + e_s,)).reshape(NW, ch, CHUNK),
                lax.slice(edge_index[1], (lo,), (lo + e_s,)).reshape(NW, ch, CHUNK),
            ],
            axis=2,
        ).reshape(NW, ch, 2 * CHUNK)
        z = _gather_sum(y, idx2)
        out = _mlp2_slice(z, W2, b2, e, s * (e_s // MLP2_BLK), out)
    return out


# speedup vs baseline: 1.0697x; 1.0235x over previous
import functools

import jax
import jax.numpy as jnp
import numpy as np
from jax import lax
from jax.experimental import pallas as pl
from jax.experimental.pallas import tpu as pltpu
from jax.experimental.pallas import tpu_sc as plsc

EPS = 0.5
NC, NS = 2, 16
NW = NC * NS
LANES = 16


def _mlp1_body(x_ref, w_ref, b_ref, y_ref):
    y_ref[...] = (
        jnp.dot(x_ref[...], w_ref[...], preferred_element_type=jnp.float32)
        + 0.5 * b_ref[...]
    )


def _mlp1(x, w1, b1):
    n, d_in = x.shape
    d_h = w1.shape[1]
    blk = 2000
    return pl.pallas_call(
        _mlp1_body,
        grid=(n // blk,),
        in_specs=[
            pl.BlockSpec((blk, d_in), lambda i: (i, 0)),
            pl.BlockSpec((d_in, d_h), lambda i: (0, 0)),
            pl.BlockSpec((1, d_h), lambda i: (0, 0)),
        ],
        out_specs=pl.BlockSpec((blk, d_h), lambda i: (i, 0)),
        out_shape=jax.ShapeDtypeStruct((n, d_h), jnp.float32),
    )(x, w1, b1.reshape(1, -1))


def _mlp2_body(z_ref, w_ref, b_ref, o_ref):
    h = jnp.maximum(z_ref[...], 0.0)
    o_ref[...] = EPS * (
        jnp.dot(h, w_ref[...], preferred_element_type=jnp.float32) + b_ref[...]
    )


def _mlp2_body_aliased(z_ref, w_ref, b_ref, oprev_ref, o_ref):
    del oprev_ref
    _mlp2_body(z_ref, w_ref, b_ref, o_ref)


MLP2_BLK = 8000


def _mlp2_slice(z, w2, b2, e_total, base_blk, out_prev):
    e_s, d_h = z.shape
    d_out = w2.shape[1]
    blk = MLP2_BLK
    in_specs = [
        pl.BlockSpec((blk, d_h), lambda i: (i, 0)),
        pl.BlockSpec((d_h, d_out), lambda i: (0, 0)),
        pl.BlockSpec((1, d_out), lambda i: (0, 0)),
    ]
    args = [z, w2, b2.reshape(1, -1)]
    kwargs = {}
    body = _mlp2_body
    if out_prev is not None:
        in_specs.append(pl.BlockSpec(memory_space=pl.ANY))
        args.append(out_prev)
        kwargs["input_output_aliases"] = {3: 0}
        body = _mlp2_body_aliased
    return pl.pallas_call(
        body,
        grid=(e_s // blk,),
        in_specs=in_specs,
        out_specs=pl.BlockSpec((blk, d_out), lambda i: (base_blk + i, 0)),
        out_shape=jax.ShapeDtypeStruct((e_total, d_out), jnp.float32),
        **kwargs,
    )(*args)


CHUNK = 40
NBUF = 3


def _gather_sum_body(chunks_per_w, d_h, e_per_w,
                     y_hbm, idx2_hbm, z_hbm,
                     idx_all, y_sh, rows0, rows1, rows2,
                     gs0, gs1, gs2, ws0, ws1, ws2):
    sid = lax.axis_index("s")
    wid = sid * NC + lax.axis_index("c")
    base = wid * e_per_w
    rows = (rows0, rows1, rows2)
    gsem = (gs0, gs1, gs2)
    wsem = (ws0, ws1, ws2)
    ch = chunks_per_w

    n_nodes = y_hbm.shape[0]
    stripe = (n_nodes // NS) & ~7
    rem = n_nodes - NS * stripe
    pltpu.sync_copy(
        y_hbm.at[pl.ds(pl.multiple_of(sid * stripe, 8), stripe)],
        y_sh.at[pl.ds(pl.multiple_of(sid * stripe, 8), stripe)],
    )
    if rem:
        @pl.when(sid == 0)
        def _stage_rem():
            pltpu.sync_copy(
                y_hbm.at[pl.ds(NS * stripe, rem)],
                y_sh.at[pl.ds(NS * stripe, rem)],
            )
    pltpu.sync_copy(idx2_hbm.at[wid], idx_all)
    plsc.subcore_barrier()

    def issue(cur, b):
        pltpu.async_copy(y_sh.at[idx_all.at[cur]], rows[b], gsem[b])

    def wait_gather(cur, b):
        pltpu.make_async_copy(y_sh.at[idx_all.at[cur]], rows[b], gsem[b]).wait()

    def wait_write(b):
        pltpu.make_async_copy(
            rows[b].at[pl.ds(0, CHUNK)], z_hbm.at[pl.ds(0, CHUNK)], wsem[b]
        ).wait()

    def process(cur, b):
        wait_gather(cur, b)

        def add_row(i, c):
            for v in range(d_h // LANES):
                sl = pl.ds(v * LANES, LANES)
                plsc.addupdate(rows[b].at[i, sl], rows[b][CHUNK + i, sl])
            return c

        plsc.parallel_loop(0, CHUNK, 1, unroll=4)(
            lambda i: add_row(i, 0) and None
        )
        off = base + cur * CHUNK
        pltpu.async_copy(
            rows[b].at[pl.ds(0, CHUNK)], z_hbm.at[pl.ds(off, CHUNK)], wsem[b]
        )

    issue(0, 0)
    issue(1, 1)

    def body(k, carry):
        g = 3 * k
        for b in range(3):
            cur = g + b
            nxt = cur + 2
            bn = (b + 2) % 3

            @pl.when(nxt < ch)
            def _issue():
                if b == 0:
                    @pl.when(k > 0)
                    def _w():
                        wait_write(bn)
                else:
                    wait_write(bn)
                issue(nxt, bn)

            process(cur, b)
        return carry

    lax.fori_loop(0, ch // 3, body, 0)
    for cur in range(3 * (ch // 3), ch):
        b = cur % 3
        nxt = cur + 2
        if nxt < ch:
            wait_write((b + 2) % 3)
            issue(nxt, (b + 2) % 3)
        process(cur, b)
    wait_write(0)
    wait_write(1)
    wait_write(2)


def _gather_sum(y, idx2):
    n, d_h = y.shape
    nw, ch, twoc = idx2.shape
    e = nw * ch * CHUNK
    e_per_w = e // NW

    mesh = plsc.VectorSubcoreMesh(
        core_axis_name="c", subcore_axis_name="s", num_cores=NC, num_subcores=NS
    )
    body = functools.partial(_gather_sum_body, ch, d_h, e_per_w)
    return pl.kernel(
        body,
        out_type=jax.ShapeDtypeStruct((e, d_h), jnp.float32),
        mesh=mesh,
        scratch_types=[
            pltpu.VMEM((ch, twoc), jnp.int32),
            pltpu.VMEM_SHARED((n, d_h), jnp.float32),
            pltpu.VMEM((2 * CHUNK, d_h), jnp.float32),
            pltpu.VMEM((2 * CHUNK, d_h), jnp.float32),
            pltpu.VMEM((2 * CHUNK, d_h), jnp.float32),
            pltpu.SemaphoreType.DMA,
            pltpu.SemaphoreType.DMA,
            pltpu.SemaphoreType.DMA,
            pltpu.SemaphoreType.DMA,
            pltpu.SemaphoreType.DMA,
            pltpu.SemaphoreType.DMA,
        ],
    )(y, idx2)


SLICE_SIZES = (96000, 128000, 96000)


def kernel(x, edge_index, W1, b1, W2, b2):
    y = _mlp1(x, W1, b1)
    e = edge_index.shape[1]
    out = None
    lo = 0
    for e_s in SLICE_SIZES:
        ch = e_s // (NW * CHUNK)
        idx2 = jnp.stack(
            [
                lax.slice(edge_index[0], (lo,), (lo + e_s,)).reshape(NW, ch, CHUNK),
                lax.slice(edge_index[1], (lo,), (lo + e_s,)).reshape(NW, ch, CHUNK),
            ],
            axis=2,
        ).reshape(NW, ch, 2 * CHUNK)
        z = _gather_sum(y, idx2)
        out = _mlp2_slice(z, W2, b2, e, lo // MLP2_BLK, out)
        lo += e_s
    return out
